# baseline (speedup 1.0000x reference)
import functools

import jax
import jax.numpy as jnp
from jax import lax
from jax.experimental import pallas as pl
from jax.experimental.pallas import tpu as pltpu
from jax.experimental.pallas import tpu_sc as plsc

_NC, _NS, _L = 2, 16, 16
_B, _C, _H, _W = 16, 64, 128, 128
_MO, _MP = 128, 17
_N = _MO * _MP
_NV = _N // _L
_CPT = _C // 2


def _sc_body(planes_hbm, tgt_hbm, ind_hbm, mask_hbm, out_hbm,
             idx_v, m_v, tv_v, pl0_v, pl1_v, pl2_v, red_v,
             sem0, sem1, sem2):
    cid = lax.axis_index("c")
    sid = lax.axis_index("s")
    g = cid * _NS + sid
    b = g // 2
    half = g % 2
    c0 = half * _CPT

    bufs = (pl0_v, pl1_v, pl2_v)
    sems = (sem0, sem1, sem2)

    pltpu.async_copy(planes_hbm.at[b, c0], pl0_v, sem0)
    pltpu.async_copy(planes_hbm.at[b, c0 + 1], pl1_v, sem1)
    pltpu.sync_copy(tgt_hbm.at[b, :, :, pl.ds(c0, _CPT)], tv_v)
    pltpu.sync_copy(ind_hbm.at[b], idx_v)
    pltpu.sync_copy(mask_hbm.at[b], m_v)

    zeros = jnp.zeros((_L,), jnp.float32)
    izeros = jnp.zeros((_L,), jnp.int32)
    iota = lax.broadcasted_iota(jnp.int32, (_L,), 0)

    def plane_acc(plane_ref, j, acc):
        jv = jnp.full((_L,), j, jnp.int32)

        def inner(i, carry):
            a, io, ip = carry
            sl = pl.ds(i * _L, _L)
            idx = idx_v[sl]
            ih = lax.shift_right_logical(idx, 7)
            iw = lax.bitwise_and(idx, 127)
            pred = plsc.load_gather(plane_ref, [ih, iw])
            t = plsc.load_gather(tv_v, [io, ip, jv])
            m = m_v[sl]
            a = a + jnp.abs(pred - t) * jnp.abs(m)
            ge = (ip >= 1).astype(jnp.int32)
            io = io + ge
            ip = ip + 16 - 17 * ge
            return (a, io, ip)

        acc, _, _ = lax.fori_loop(0, _NV, inner, (acc, izeros, iota),
                                  unroll=8)
        return acc

    def step(jj, acc):
        for k in range(3):
            j = 3 * jj + k
            buf, sem = bufs[k], sems[k]
            pltpu.make_async_copy(planes_hbm.at[b, c0], buf, sem).wait()

            @pl.when(j + 2 < _CPT)
            def _(j=j, k=k):
                pltpu.async_copy(planes_hbm.at[b, c0 + j + 2],
                                 bufs[(k + 2) % 3], sems[(k + 2) % 3])

            acc = plane_acc(buf, j, acc)
        return acc

    acc = lax.fori_loop(0, _CPT // 3, step, zeros)
    for j in range(3 * (_CPT // 3), _CPT):
        k = j % 3
        pltpu.make_async_copy(planes_hbm.at[b, c0], bufs[k], sems[k]).wait()
        acc = plane_acc(bufs[k], j, acc)

    def msum_step(i, a):
        return a + m_v[pl.ds(i * _L, _L)]

    msum = lax.fori_loop(0, _NV, msum_step, zeros)
    msum = msum * (half == 0).astype(jnp.float32)

    red_v[0, :] = acc
    red_v[1, :] = msum
    pltpu.sync_copy(red_v, out_hbm.at[cid, sid])


@functools.cache
def _sc_kernel():
    return functools.partial(
        pl.kernel,
        out_type=jax.ShapeDtypeStruct((_NC, _NS, 2, _L), jnp.float32),
        mesh=plsc.VectorSubcoreMesh(
            core_axis_name="c", subcore_axis_name="s",
            num_cores=_NC, num_subcores=_NS),
        compiler_params=pltpu.CompilerParams(
            needs_layout_passes=False, use_tc_tiling_on_sc=False),
        scratch_types=[
            pltpu.VMEM((_N,), jnp.int32),
            pltpu.VMEM((_N,), jnp.float32),
            pltpu.VMEM((_MO, _MP, _CPT), jnp.float32),
            pltpu.VMEM((_H, _W), jnp.float32),
            pltpu.VMEM((_H, _W), jnp.float32),
            pltpu.VMEM((_H, _W), jnp.float32),
            pltpu.VMEM((2, _L), jnp.float32),
            pltpu.SemaphoreType.DMA,
            pltpu.SemaphoreType.DMA,
            pltpu.SemaphoreType.DMA,
        ],
    )(_sc_body)


def kernel(output, target, ind, ind_mask):
    b, C, H, W = output.shape
    parts = _sc_kernel()(output, target,
                         ind.reshape(b, _N), ind_mask.reshape(b, _N))
    abs_sum = jnp.sum(parts[:, :, 0, :])
    mask_sum = jnp.sum(parts[:, :, 1, :])
    return abs_sum / (C * mask_sum + 0.0001)
